# Initial kernel scaffold; baseline (speedup 1.0000x reference)
import jax
import jax.numpy as jnp
from jax.experimental import pallas as pl


def kernel(x, w1, b1, w2, b2, w3, b3, w4, b4):
    raise NotImplementedError("write your pallas kernel here")


# speedup vs baseline: 1.7993x; 1.7993x over previous
import jax
import jax.numpy as jnp
from jax.experimental import pallas as pl
from jax.experimental.pallas import tpu as pltpu

G = 8
TB = 4096


def _mlp_packed_kernel(x_ref, w1_ref, b1_ref, w2_ref, b2_ref, w3_ref,
                       b3_ref, w4_ref, b4_ref, o_ref):
    h = x_ref[...].astype(jnp.bfloat16)
    h = jnp.dot(h, w1_ref[...], preferred_element_type=jnp.float32)
    h = jnp.maximum(h + b1_ref[...], 0.0).astype(jnp.bfloat16)
    h = jnp.dot(h, w2_ref[...], preferred_element_type=jnp.float32)
    h = jnp.maximum(h + b2_ref[...], 0.0).astype(jnp.bfloat16)
    h = jnp.dot(h, w3_ref[...], preferred_element_type=jnp.float32)
    h = jnp.maximum(h + b3_ref[...], 0.0).astype(jnp.bfloat16)
    h = jnp.dot(h, w4_ref[...], preferred_element_type=jnp.float32)
    o_ref[...] = jax.nn.sigmoid(h + b4_ref[...])


def kernel(x, w1, b1, w2, b2, w3, b3, w4, b4):
    B, f_in = x.shape
    n_out = w4.shape[1]

    xp = x.reshape(B // G, f_in * G)

    eye = jnp.eye(G, dtype=jnp.float32)

    def pack_w(w):
        return jnp.kron(eye, w).astype(jnp.bfloat16)

    def pack_b(b):
        return jnp.tile(b, (1, G))

    w1p, b1p = pack_w(w1), pack_b(b1)
    w2p, b2p = pack_w(w2), pack_b(b2)
    w3p, b3p = pack_w(w3), pack_b(b3)
    w4p, b4p = pack_w(w4), pack_b(b4)

    Bp = B // G
    tb = min(TB, Bp)
    n_blocks = pl.cdiv(Bp, tb)

    def const(arr):
        return pl.BlockSpec(arr.shape, lambda i: (0,) * arr.ndim)

    outp = pl.pallas_call(
        _mlp_packed_kernel,
        out_shape=jax.ShapeDtypeStruct((Bp, n_out * G), jnp.float32),
        grid=(n_blocks,),
        in_specs=[pl.BlockSpec((tb, f_in * G), lambda i: (i, 0)),
                  const(w1p), const(b1p),
                  const(w2p), const(b2p),
                  const(w3p), const(b3p),
                  const(w4p), const(b4p)],
        out_specs=pl.BlockSpec((tb, n_out * G), lambda i: (i, 0)),
        compiler_params=pltpu.CompilerParams(
            dimension_semantics=("parallel",),
            vmem_limit_bytes=48 * 1024 * 1024,
        ),
    )(xp, w1p, b1p, w2p, b2p, w3p, b3p, w4p, b4p)

    return outp.reshape(B, n_out)
